# Initial kernel scaffold; baseline (speedup 1.0000x reference)
import jax
import jax.numpy as jnp
from jax.experimental import pallas as pl


def kernel(h, pe, x, t, context, edges, edge_index, edge_attr, batch, params):
    raise NotImplementedError("write your pallas kernel here")


# speedup vs baseline: 2.6741x; 2.6741x over previous
import functools
import math

import jax
import jax.numpy as jnp
import numpy as np
from jax import lax
from jax.experimental import pallas as pl
from jax.experimental.pallas import tpu as pltpu
from jax.experimental.pallas import tpu_sc as plsc

F32 = jnp.float32
I32 = jnp.int32

N = 50000
E = 800001
G = 512
N_PAD = 51200
E_PAD = 819200
BN_ = 1024
NGRID = 50
BE = 2048
EGRID = 400
NC, NS = 2, 16
EW = E_PAD // (NC * NS)
NG_G = EW // 1024
EWS = E_PAD // NS
CH_S = 1024
NCH_S = EWS // CH_S
RPT = N_PAD // NS

_FREQS = np.exp(-math.log(10000.0) * np.arange(4, dtype=np.float32) / 4.0
                ).reshape(1, 4)


def _rep8(v):
    return jnp.broadcast_to(v.reshape(1, -1), (8, v.shape[-1]))


def _padrows(w, r):
    return jnp.concatenate(
        [w, jnp.zeros((r - w.shape[0], w.shape[1]), F32)], axis=0)


def _stats_body(peb, osum, osq):
    @pl.when(pl.program_id(0) == 0)
    def _():
        osum[...] = jnp.zeros_like(osum)
        osq[...] = jnp.zeros_like(osq)
    v = peb[...]
    s = jnp.sum(v, axis=0, keepdims=True)
    q = jnp.sum(v * v, axis=0, keepdims=True)
    osum[...] += jnp.broadcast_to(s, osum.shape)
    osq[...] += jnp.broadcast_to(q, osq.shape)


def _prep_body(hb, peb, tb, ctxb, ssum, ssq,
               node_W, node_b, pe_W, pe_b, bn_w, bn_b, ctx_W, ctx_b,
               inW, inb, A1, B1, C21, C22,
               o_h1, o_TA, o_TB, o_tc1, o_tc2):
    mean = ssum[0:1] * (1.0 / N)
    var = ssq[0:1] * (1.0 / N) - mean * mean
    hpe = ((peb[...] - mean) / jnp.sqrt(var + 1e-5)) * bn_w[0:1] + bn_b[0:1]
    fio = jax.lax.broadcasted_iota(I32, (1, 4), 1).astype(F32)
    freqs = jnp.exp(fio * (-math.log(10000.0) / 4.0))
    targ = tb[...] * freqs
    te = jnp.concatenate([jnp.cos(targ), jnp.sin(targ)], axis=1)
    hc = jnp.concatenate([
        jnp.dot(hb[...], node_W[...]) + node_b[0:1],
        jnp.dot(hpe, pe_W[...]) + pe_b[0:1],
        te,
        jnp.dot(ctxb[...], ctx_W[...]) + ctx_b[0:1],
    ], axis=1)
    h1 = jnp.dot(hc, inW[...]) + inb[0:1]
    o_h1[...] = h1
    o_TA[...] = jnp.dot(h1, A1[...])
    o_TB[...] = jnp.dot(h1, B1[...])
    o_tc1[...] = jnp.dot(te, C21[...])
    o_tc2[...] = jnp.dot(te, C22[...])


def _edge_body(gab, xr, xc, eab, teC, teC0, wr, W4, bconst, e2W, e2b,
               o_lo, o_hi):
    i = pl.program_id(0)
    d = xr[...] - xc[...]
    radial = jnp.sum(d * d, axis=1, keepdims=True)
    rsel = (jax.lax.broadcasted_iota(I32, (BE, 128), 0) // 16
            == jax.lax.broadcasted_iota(I32, (BE, 128), 1)).astype(F32)
    terep = jnp.dot(rsel, teC[...])
    gid = i * BE + jax.lax.broadcasted_iota(I32, (BE, 1), 0)
    terep = jnp.where(gid == E - 1, teC0[0:1], terep)
    z = (gab[...] + radial * wr[0:1] + jnp.dot(eab[...], W4[...])
         + terep + bconst[0:1])
    m = z * jax.nn.sigmoid(z)
    m2 = jnp.dot(m, e2W[...]) + e2b[0:1]
    m2 = m2 * jax.nn.sigmoid(m2)
    o_lo[...] = m2[:, :32]
    o_hi[...] = m2[:, 32:]


def _node1_body(h1b, a0, a1, n1h, n1a0, n1a1, n1b, n2W, n2b, outW, outb,
                inW2, inb2, A2, B2, o_h12, o_TA2, o_TB2):
    h1 = h1b[...]
    u = (jnp.dot(h1, n1h[...]) + jnp.dot(a0[...], n1a0[...])
         + jnp.dot(a1[...], n1a1[...]) + n1b[0:1])
    u = u * jax.nn.sigmoid(u)
    h1n = h1 + jnp.dot(u, n2W[...]) + n2b[0:1]
    hc2 = jnp.dot(h1n, outW[...]) + outb[0:1]
    h12 = jnp.dot(hc2, inW2[...]) + inb2[0:1]
    o_h12[...] = h12
    o_TA2[...] = jnp.dot(h12, A2[...])
    o_TB2[...] = jnp.dot(h12, B2[...])


def _node2_body(h1b, a0, a1, n1h, n1a0, n1a1, n1b, n2W, n2b, outW, outb,
                o_hc):
    h1 = h1b[...]
    u = (jnp.dot(h1, n1h[...]) + jnp.dot(a0[...], n1a0[...])
         + jnp.dot(a1[...], n1a1[...]) + n1b[0:1])
    u = u * jax.nn.sigmoid(u)
    h1n = h1 + jnp.dot(u, n2W[...]) + n2b[0:1]
    o_hc[...] = jnp.dot(h1n, outW[...]) + outb[0:1]


def _pool_body(hcb, bb, m1W, m1b, m2W, m2b, m3W, m3b, out, acc):
    i = pl.program_id(0)
    @pl.when(i == 0)
    def _():
        acc[...] = jnp.zeros_like(acc)
    sel = (bb[0] == jax.lax.broadcasted_iota(I32, (G, BN_), 0)).astype(F32)
    acc[...] += jnp.dot(sel, hcb[...])
    @pl.when(i == NGRID - 1)
    def _():
        z = jnp.maximum(jnp.dot(acc[...], m1W[...]) + m1b[0:1], 0.0)
        z = jnp.maximum(jnp.dot(z, m2W[...]) + m2b[0:1], 0.0)
        out[...] = jnp.dot(z, m3W[...]) + m3b[0:1]


def _full_spec(shape):
    nd = len(shape)
    return pl.BlockSpec(shape, lambda i: (0,) * nd)


def _nblk_spec(k):
    return pl.BlockSpec((BN_, k), lambda i: (i, 0))


def _eblk_spec(k):
    return pl.BlockSpec((BE, k), lambda i: (i, 0))


def _stats_call(pe_p):
    return pl.pallas_call(
        _stats_body,
        grid=(NGRID,),
        in_specs=[_nblk_spec(24)],
        out_specs=[_full_spec((8, 24)), _full_spec((8, 24))],
        out_shape=[jax.ShapeDtypeStruct((8, 24), F32)] * 2,
    )(pe_p)


def _prep_call(h_p, pe_p, t_p, ctx_p, ssum, ssq, weights):
    nspec = [_nblk_spec(64), _nblk_spec(24), _nblk_spec(1), _nblk_spec(64)]
    wspec = [_full_spec(w.shape) for w in (ssum, ssq) + weights]
    return pl.pallas_call(
        _prep_body,
        grid=(NGRID,),
        in_specs=nspec + wspec,
        out_specs=[_nblk_spec(64)] * 5,
        out_shape=[jax.ShapeDtypeStruct((N_PAD, 64), F32)] * 5,
    )(h_p, pe_p, t_p, ctx_p, ssum, ssq, *weights)


def _edge_call(gab, xr, xc, ea_p, teC, weights):
    especs = [_eblk_spec(64), _eblk_spec(16), _eblk_spec(16), _eblk_spec(8),
              pl.BlockSpec((128, 64), lambda i: (i, 0))]
    wspec = [_full_spec(w.shape) for w in weights]
    return pl.pallas_call(
        _edge_body,
        grid=(EGRID,),
        in_specs=especs + wspec,
        out_specs=[_eblk_spec(32)] * 2,
        out_shape=[jax.ShapeDtypeStruct((E_PAD, 32), F32)] * 2,
    )(gab, xr, xc, ea_p, teC, *weights)


def _node1_call(h1, a0, a1, weights):
    specs = [_nblk_spec(64), _nblk_spec(32), _nblk_spec(32)]
    wspec = [_full_spec(w.shape) for w in weights]
    return pl.pallas_call(
        _node1_body,
        grid=(NGRID,),
        in_specs=specs + wspec,
        out_specs=[_nblk_spec(64)] * 3,
        out_shape=[jax.ShapeDtypeStruct((N_PAD, 64), F32)] * 3,
    )(h1, a0, a1, *weights)


def _node2_call(h1, a0, a1, weights):
    specs = [_nblk_spec(64), _nblk_spec(32), _nblk_spec(32)]
    wspec = [_full_spec(w.shape) for w in weights]
    return pl.pallas_call(
        _node2_body,
        grid=(NGRID,),
        in_specs=specs + wspec,
        out_specs=[_nblk_spec(64)],
        out_shape=[jax.ShapeDtypeStruct((N_PAD, 64), F32)],
    )(h1, a0, a1, *weights)


def _pool_call(hc, batch3, weights):
    specs = [_nblk_spec(64), pl.BlockSpec((1, 1, BN_), lambda i: (i, 0, 0))]
    wspec = [_full_spec(w.shape) for w in weights]
    return pl.pallas_call(
        _pool_body,
        grid=(NGRID,),
        in_specs=specs + wspec,
        out_specs=[_full_spec((G, 8))],
        out_shape=[jax.ShapeDtypeStruct((G, 8), F32)],
        scratch_shapes=[pltpu.VMEM((G, 64), F32)],
    )(hc, batch3, *weights)


def _sc_mesh():
    return plsc.VectorSubcoreMesh(core_axis_name="c", subcore_axis_name="s",
                                  num_cores=NC, num_subcores=NS)


def _gather1(ta, tb, x16, rows2, cols2):
    outs = [jax.ShapeDtypeStruct((E_PAD, 64), F32),
            jax.ShapeDtypeStruct((E_PAD, 16), F32),
            jax.ShapeDtypeStruct((E_PAD, 16), F32)]
    scratch = [pltpu.VMEM((8, 128), I32), pltpu.VMEM((8, 128), I32),
               pltpu.VMEM((512, 64), F32), pltpu.VMEM((512, 64), F32),
               pltpu.VMEM((512, 16), F32), pltpu.VMEM((512, 16), F32),
               pltpu.SemaphoreType.DMA]

    @functools.partial(pl.kernel, out_type=outs, mesh=_sc_mesh(),
                       scratch_types=scratch,
                       compiler_params=pltpu.CompilerParams(
                           use_tc_tiling_on_sc=False))
    def gk(ta_h, tb_h, x_h, r_h, c_h, gab_o, xr_o, xc_o,
           idxr, idxc, bufA, bufB, bxr, bxc, sem):
        wid = lax.axis_index("s") * NC + lax.axis_index("c")
        base = wid * EW
        g0 = wid * NG_G

        def group(j, carry):
            pltpu.sync_copy(r_h.at[g0 + j], idxr)
            pltpu.sync_copy(c_h.at[g0 + j], idxc)
            for half in range(2):
                e0 = base + j * 1024 + half * 512
                descs = []
                for q in range(4):
                    iq = half * 4 + q
                    sl = pl.ds(q * 128, 128)
                    descs.append(pltpu.async_copy(
                        ta_h.at[idxr.at[iq]], bufA.at[sl], sem))
                    descs.append(pltpu.async_copy(
                        tb_h.at[idxc.at[iq]], bufB.at[sl], sem))
                    descs.append(pltpu.async_copy(
                        x_h.at[idxr.at[iq]], bxr.at[sl], sem))
                    descs.append(pltpu.async_copy(
                        x_h.at[idxc.at[iq]], bxc.at[sl], sem))
                for dsc in descs:
                    dsc.wait()

                def addrow(r, c2):
                    for k in range(4):
                        cs = pl.ds(k * 16, 16)
                        bufA[r, cs] = bufA[r, cs] + bufB[r, cs]
                    return c2
                lax.fori_loop(0, 512, addrow, 0)
                pltpu.sync_copy(bufA, gab_o.at[pl.ds(e0, 512)])
                pltpu.sync_copy(bxr, xr_o.at[pl.ds(e0, 512)])
                pltpu.sync_copy(bxc, xc_o.at[pl.ds(e0, 512)])
            return carry

        lax.fori_loop(0, NG_G, group, 0)

    return gk(ta, tb, x16, rows2, cols2)


def _gather2(ta, tb, rows2, cols2):
    outs = [jax.ShapeDtypeStruct((E_PAD, 64), F32)]
    scratch = [pltpu.VMEM((8, 128), I32), pltpu.VMEM((8, 128), I32),
               pltpu.VMEM((512, 64), F32), pltpu.VMEM((512, 64), F32),
               pltpu.SemaphoreType.DMA]

    @functools.partial(pl.kernel, out_type=outs, mesh=_sc_mesh(),
                       scratch_types=scratch,
                       compiler_params=pltpu.CompilerParams(
                           use_tc_tiling_on_sc=False))
    def gk(ta_h, tb_h, r_h, c_h, gab_o, idxr, idxc, bufA, bufB, sem):
        wid = lax.axis_index("s") * NC + lax.axis_index("c")
        base = wid * EW
        g0 = wid * NG_G

        def group(j, carry):
            pltpu.sync_copy(r_h.at[g0 + j], idxr)
            pltpu.sync_copy(c_h.at[g0 + j], idxc)
            for half in range(2):
                e0 = base + j * 1024 + half * 512
                descs = []
                for q in range(4):
                    iq = half * 4 + q
                    sl = pl.ds(q * 128, 128)
                    descs.append(pltpu.async_copy(
                        ta_h.at[idxr.at[iq]], bufA.at[sl], sem))
                    descs.append(pltpu.async_copy(
                        tb_h.at[idxc.at[iq]], bufB.at[sl], sem))
                for dsc in descs:
                    dsc.wait()

                def addrow(r, c2):
                    for k in range(4):
                        cs = pl.ds(k * 16, 16)
                        bufA[r, cs] = bufA[r, cs] + bufB[r, cs]
                    return c2
                lax.fori_loop(0, 512, addrow, 0)
                pltpu.sync_copy(bufA, gab_o.at[pl.ds(e0, 512)])
            return carry

        lax.fori_loop(0, NG_G, group, 0)

    return gk(ta, tb, rows2, cols2)[0]


def _scatter(m2lo, m2hi, rows2, zrows):
    outs = [jax.ShapeDtypeStruct((2, N_PAD, 32), F32)]
    scratch = [pltpu.VMEM((8, 128), I32), pltpu.VMEM((512, 32), F32),
               pltpu.VMEM_SHARED((N_PAD, 32), F32)]

    @functools.partial(pl.kernel, out_type=outs, mesh=_sc_mesh(),
                       scratch_types=scratch,
                       compiler_params=pltpu.CompilerParams(
                           use_tc_tiling_on_sc=False))
    def sk(lo_h, hi_h, r_h, z_h, agg_o, idxb, valb, acc):
        cid = lax.axis_index("c")
        sid = lax.axis_index("s")
        r0 = sid * RPT
        pltpu.sync_copy(z_h, acc.at[pl.ds(r0, RPT)])
        plsc.subcore_barrier()
        base = sid * EWS
        g0 = base // 1024

        def chunk(j, carry):
            pltpu.sync_copy(r_h.at[g0 + j], idxb)
            for half in range(2):
                e0 = base + j * CH_S + half * 512

                @pl.when(cid == 0)
                def _():
                    pltpu.sync_copy(lo_h.at[pl.ds(e0, 512)], valb)

                @pl.when(cid == 1)
                def _():
                    pltpu.sync_copy(hi_h.at[pl.ds(e0, 512)], valb)

                for q in range(4):
                    pltpu.sync_copy(valb.at[pl.ds(q * 128, 128)],
                                    acc.at[idxb.at[half * 4 + q]], add=True)
            return carry

        lax.fori_loop(0, NCH_S, chunk, 0)
        plsc.subcore_barrier()
        pltpu.sync_copy(acc.at[pl.ds(r0, RPT)],
                        agg_o.at[cid, pl.ds(r0, RPT)])

    return sk(m2lo, m2hi, rows2, zrows)[0]


def kernel(h, pe, x, t, context, edges, edge_index, edge_attr, batch, params):
    p = params
    convs = p['convs']

    znp = lambda r, k: jnp.zeros((r, k), F32)
    h_p = jnp.concatenate([h, znp(N_PAD - N, 64)], 0)
    pe_p = jnp.concatenate(
        [jnp.concatenate([pe, znp(N, 4)], 1), znp(N_PAD - N, 24)], 0)
    t_p = jnp.concatenate([t, jnp.zeros((N_PAD - N,), F32)], 0).reshape(
        N_PAD, 1)
    ctx_p = jnp.concatenate([context, znp(N_PAD - N, 64)], 0)
    x16 = jnp.concatenate(
        [jnp.concatenate([x, znp(N, 13)], 1), znp(N_PAD - N, 16)], 0)
    ea_p = jnp.concatenate(
        [jnp.concatenate([edge_attr, znp(E, 4)], 1), znp(E_PAD - E, 8)], 0)
    rows2 = jnp.concatenate(
        [edges[0], jnp.full((E_PAD - E,), N, I32)]).reshape(
            E_PAD // 1024, 8, 128)
    cols2 = jnp.concatenate(
        [edges[1], jnp.full((E_PAD - E,), N, I32)]).reshape(
            E_PAD // 1024, 8, 128)
    batch3 = jnp.concatenate(
        [batch, jnp.full((N_PAD - N,), G, I32)]).reshape(NGRID, 1, BN_)
    zrows = znp(RPT, 32)

    def conv_parts(c):
        e1 = c['e1_W']
        A, B = e1[:64], e1[64:128]
        wr = _rep8(e1[128])
        C1, C2 = e1[129:185], e1[185:193]
        W4 = _padrows(p['edge_W'] @ C1, 8)
        bconst = _rep8(p['edge_b'] @ C1 + c['e1_b'])
        return A, B, wr, C1, C2, W4, bconst

    A1, B1, wr1, _, C21, W41, bc1 = conv_parts(convs[0])
    A2, B2, wr2, _, C22, W42, bc2 = conv_parts(convs[1])

    prep_w = (p['node_W'], _rep8(p['node_b']), _padrows(p['pe_W'], 24),
              _rep8(p['pe_b']), _rep8(jnp.concatenate([p['bn_w'],
                                                       jnp.zeros(4, F32)])),
              _rep8(jnp.concatenate([p['bn_b'], jnp.zeros(4, F32)])),
              p['ctx_W'], _rep8(p['ctx_b']),
              convs[0]['in_W'], _rep8(convs[0]['in_b']), A1, B1, C21, C22)

    def node_w(c):
        n1 = c['n1_W']
        return (n1[:64], n1[64:96], n1[96:128], _rep8(c['n1_b']),
                c['n2_W'], _rep8(c['n2_b']), c['out_W'], _rep8(c['out_b']))

    node1_w = node_w(convs[0]) + (convs[1]['in_W'], _rep8(convs[1]['in_b']),
                                  A2, B2)
    node2_w = node_w(convs[1])

    pool_w = (p['m1_W'], _rep8(p['m1_b']), p['m2_W'], _rep8(p['m2_b']),
              jnp.concatenate([p['m3_W'], jnp.zeros((16, 7), F32)], 1),
              _rep8(jnp.concatenate([p['m3_b'], jnp.zeros(7, F32)])))

    ssum, ssq = _stats_call(pe_p)
    h1_1, TA1, TB1, teC1, teC2 = _prep_call(h_p, pe_p, t_p, ctx_p,
                                            ssum, ssq, prep_w)

    gab1, xr, xc = _gather1(TA1, TB1, x16, rows2, cols2)
    teC1_0 = jnp.broadcast_to(teC1[0:1], (8, 64))
    m2lo, m2hi = _edge_call(gab1, xr, xc, ea_p, teC1,
                            (teC1_0, wr1, W41, bc1,
                             convs[0]['e2_W'], _rep8(convs[0]['e2_b'])))
    agg1 = _scatter(m2lo, m2hi, rows2, zrows)
    h1_2, TA2, TB2 = _node1_call(h1_1, agg1[0], agg1[1], node1_w)

    gab2 = _gather2(TA2, TB2, rows2, cols2)
    teC2_0 = jnp.broadcast_to(teC2[0:1], (8, 64))
    m2lo2, m2hi2 = _edge_call(gab2, xr, xc, ea_p, teC2,
                              (teC2_0, wr2, W42, bc2,
                               convs[1]['e2_W'], _rep8(convs[1]['e2_b'])))
    agg2 = _scatter(m2lo2, m2hi2, rows2, zrows)
    hcF = _node2_call(h1_2, agg2[0], agg2[1], node2_w)[0]

    out8 = _pool_call(hcF, batch3, pool_w)[0]
    return out8[:, :1]
